# baseline (speedup 1.0000x reference)
import functools

import jax
import jax.numpy as jnp
from jax import lax
from jax.experimental import pallas as pl
from jax.experimental.pallas import tpu as pltpu
from jax.experimental.pallas import tpu_sc as plsc

_NC = 2
_NS = 16
_L = 16


def _sc_pos_kernel(h, w, d, b, row_hbm, col_hbm, out_hbm, tab_v, buf_v, sem):
    hw = h * w
    ch_per_w = 2 * d // (_NC * _NS)
    wid = lax.axis_index("s") * _NC + lax.axis_index("c")
    half = d // ch_per_w
    is_ye = wid >= half
    c_loc = (wid % half) * ch_per_w

    @pl.when(jnp.logical_not(is_ye))
    def _():
        pltpu.sync_copy(col_hbm, tab_v)

    @pl.when(is_ye)
    def _():
        pltpu.sync_copy(row_hbm, tab_v)

    lane = lax.iota(jnp.int32, _L)
    a_mul = jnp.where(is_ye, w, 1).astype(jnp.int32)
    b_mul = jnp.where(is_ye, 1, w).astype(jnp.int32)

    lane_base = lane * hw

    def fill_row(r, carry):
        v = tab_v[r, pl.ds(c_loc, ch_per_w)]
        base = lane_base + r * a_mul
        for t in range(w):
            plsc.store_scatter(buf_v, [base + t * b_mul], v)
        return carry

    lax.fori_loop(0, h, fill_row, 0)

    c_out = wid * ch_per_w * hw
    handles = [
        pltpu.async_copy(buf_v, out_hbm.at[i, pl.ds(c_out, ch_per_w * hw)], sem)
        for i in range(b)
    ]
    for hnd in handles:
        hnd.wait()


def kernel(input_tensor, row_embed, col_embed):
    b = input_tensor.shape[0]
    h, w = input_tensor.shape[-2], input_tensor.shape[-1]
    d = row_embed.shape[-1]
    row = lax.slice(row_embed, (0, 0), (h, d))
    col = lax.slice(col_embed, (0, 0), (w, d))
    ch_per_w = 2 * d // (_NC * _NS)
    mesh = plsc.VectorSubcoreMesh(core_axis_name="c", subcore_axis_name="s")
    f = pl.kernel(
        functools.partial(_sc_pos_kernel, h, w, d, b),
        out_type=jax.ShapeDtypeStruct((b, 2 * d * h * w), jnp.float32),
        mesh=mesh,
        scratch_types=[
            pltpu.VMEM((h, d), jnp.float32),
            pltpu.VMEM((ch_per_w * h * w,), jnp.float32),
            pltpu.SemaphoreType.DMA,
        ],
        compiler_params=pltpu.CompilerParams(
            use_tc_tiling_on_sc=False, needs_layout_passes=False
        ),
    )
    out = f(row, col)
    return out.reshape(b, 2 * d, h, w)


# speedup vs baseline: 1.0069x; 1.0069x over previous
import functools

import jax
import jax.numpy as jnp
from jax import lax
from jax.experimental import pallas as pl
from jax.experimental.pallas import tpu as pltpu
from jax.experimental.pallas import tpu_sc as plsc

_NC = 2
_NS = 16
_L = 16


def _sc_pos_kernel(h, w, d, b, row_hbm, col_hbm, out_hbm, tab_v, buf_v, sem):
    hw = h * w
    ch_per_w = 2 * d // (_NC * _NS)
    wid = lax.axis_index("s") * _NC + lax.axis_index("c")
    half = d // ch_per_w
    is_ye = wid >= half
    c_loc = (wid % half) * ch_per_w

    @pl.when(jnp.logical_not(is_ye))
    def _():
        pltpu.sync_copy(col_hbm, tab_v)

    @pl.when(is_ye)
    def _():
        pltpu.sync_copy(row_hbm, tab_v)

    lane = lax.iota(jnp.int32, _L)
    a_mul = jnp.where(is_ye, w, 1).astype(jnp.int32)
    b_mul = jnp.where(is_ye, 1, w).astype(jnp.int32)

    lane_base = lane * hw

    def fill_row(r, carry):
        v = tab_v[r, pl.ds(c_loc, ch_per_w)]
        base = lane_base + r * a_mul
        for t in range(w):
            plsc.store_scatter(buf_v, [base + t * b_mul], v)
        return carry

    lax.fori_loop(0, h, fill_row, 0)

    c_out = wid * ch_per_w * hw
    handles = [
        pltpu.async_copy(
            buf_v, out_hbm.at[pl.ds(i * 2 * d * hw + c_out, ch_per_w * hw)], sem
        )
        for i in range(b)
    ]
    for hnd in handles:
        hnd.wait()


def kernel(input_tensor, row_embed, col_embed):
    b = input_tensor.shape[0]
    h, w = input_tensor.shape[-2], input_tensor.shape[-1]
    d = row_embed.shape[-1]
    row = lax.slice(row_embed, (0, 0), (h, d))
    col = lax.slice(col_embed, (0, 0), (w, d))
    ch_per_w = 2 * d // (_NC * _NS)
    mesh = plsc.VectorSubcoreMesh(core_axis_name="c", subcore_axis_name="s")
    f = pl.kernel(
        functools.partial(_sc_pos_kernel, h, w, d, b),
        out_type=jax.ShapeDtypeStruct((b * 2 * d * h * w,), jnp.float32),
        mesh=mesh,
        scratch_types=[
            pltpu.VMEM((h, d), jnp.float32),
            pltpu.VMEM((ch_per_w * h * w,), jnp.float32),
            pltpu.SemaphoreType.DMA,
        ],
        compiler_params=pltpu.CompilerParams(needs_layout_passes=False),
    )
    out = f(row, col)
    return out.reshape(b, 2 * d, h, w)


# speedup vs baseline: 2.5609x; 2.5434x over previous
import functools

import jax
import jax.numpy as jnp
from jax import lax
from jax.experimental import pallas as pl
from jax.experimental.pallas import tpu as pltpu
from jax.experimental.pallas import tpu_sc as plsc

_NC = 2
_NS = 16
_L = 16


def _sc_pos_kernel(h, w, d, b, row_hbm, col_hbm, out_hbm, col_v, row_v, buf_v, sem):
    wid = lax.axis_index("s") * _NC + lax.axis_index("c")

    pltpu.sync_copy(col_hbm, col_v)
    pltpu.sync_copy(row_hbm.at[pl.ds(wid * d, d)], row_v)

    rvecs = [row_v[pl.ds(k * _L, _L)] for k in range(d // _L)]
    for wi in range(w):
        for k in range(d // _L):
            buf_v[wi, pl.ds(k * _L, _L)] = col_v[pl.ds(wi * d + k * _L, _L)]
        for k in range(d // _L):
            buf_v[wi, pl.ds(d + k * _L, _L)] = rvecs[k]

    handles = [
        pltpu.async_copy(buf_v, out_hbm.at[i, wid], sem) for i in range(b)
    ]
    for hnd in handles:
        hnd.wait()


def kernel(input_tensor, row_embed, col_embed):
    b = input_tensor.shape[0]
    h, w = input_tensor.shape[-2], input_tensor.shape[-1]
    d = row_embed.shape[-1]
    row = lax.slice(row_embed, (0, 0), (h, d)).reshape(h * d)
    col = lax.slice(col_embed, (0, 0), (w, d)).reshape(w * d)
    mesh = plsc.VectorSubcoreMesh(core_axis_name="c", subcore_axis_name="s")
    f = pl.kernel(
        functools.partial(_sc_pos_kernel, h, w, d, b),
        out_type=jax.ShapeDtypeStruct((b, h, w, 2 * d), jnp.float32),
        mesh=mesh,
        scratch_types=[
            pltpu.VMEM((w * d,), jnp.float32),
            pltpu.VMEM((d,), jnp.float32),
            pltpu.VMEM((w, 2 * d), jnp.float32),
            pltpu.SemaphoreType.DMA,
        ],
        compiler_params=pltpu.CompilerParams(needs_layout_passes=False),
    )
    out = f(row, col)
    return out.transpose(0, 3, 1, 2)
